# Initial kernel scaffold; baseline (speedup 1.0000x reference)
import jax
import jax.numpy as jnp
from jax.experimental import pallas as pl


def kernel(node_feature, adjacency_matrix, W1, b1, W2, b2):
    raise NotImplementedError("write your pallas kernel here")


# speedup vs baseline: 1683.8054x; 1683.8054x over previous
import jax
import jax.numpy as jnp
from jax.experimental import pallas as pl

_N = 1024


def _gcn2_kernel(a_ref, x_ref, w1_ref, b1_ref, w2_ref, b2_ref, out_ref):
    af = (a_ref[...] != 0).astype(jnp.float32)
    ones = jnp.ones((_N, 1), jnp.float32)
    deg = jax.lax.dot_general(
        af, ones, (((0,), (0,)), ((), ())),
        preferred_element_type=jnp.float32,
        precision=jax.lax.Precision.HIGHEST,
    ) + 1.0
    dinv = jax.lax.rsqrt(deg)
    dinv2 = dinv * dinv

    def prop(h, b):
        hm = h * dinv
        agg = jax.lax.dot_general(
            af, hm, (((0,), (0,)), ((), ())),
            preferred_element_type=jnp.float32,
            precision=jax.lax.Precision.HIGHEST,
        )
        return dinv * agg + dinv2 * h + b

    h1 = jnp.dot(x_ref[...], w1_ref[...],
                 preferred_element_type=jnp.float32,
                 precision=jax.lax.Precision.HIGHEST)
    y1 = jax.nn.relu(prop(h1, b1_ref[...]))
    h2 = jnp.dot(y1, w2_ref[...],
                 preferred_element_type=jnp.float32,
                 precision=jax.lax.Precision.HIGHEST)
    out_ref[...] = prop(h2, b2_ref[...])


def kernel(node_feature, adjacency_matrix, W1, b1, W2, b2):
    x = node_feature.astype(jnp.float32)
    if x.ndim == 3:
        x = x.reshape(-1, x.shape[-1])
    n = x.shape[0]
    out = pl.pallas_call(
        _gcn2_kernel,
        out_shape=jax.ShapeDtypeStruct((n, W2.shape[1]), jnp.float32),
    )(adjacency_matrix, x, W1, b1.reshape(1, -1), W2, b2.reshape(1, -1))
    return out.reshape(1, n, W2.shape[1])


# speedup vs baseline: 5467.5350x; 3.2471x over previous
import jax
import jax.numpy as jnp
from jax.experimental import pallas as pl

_N = 1024


def _gcn2_kernel(a_ref, x_ref, w1_ref, b1_ref, w2_ref, b2_ref, out_ref):
    af = (a_ref[...] != 0).astype(jnp.float32)
    ones = jnp.ones((_N, 1), jnp.float32)
    deg = jax.lax.dot_general(
        af, ones, (((0,), (0,)), ((), ())),
        preferred_element_type=jnp.float32,
    ) + 1.0
    dinv = jax.lax.rsqrt(deg)
    dinv2 = dinv * dinv

    def prop(h, b):
        hm = h * dinv
        agg = jax.lax.dot_general(
            af, hm, (((0,), (0,)), ((), ())),
            preferred_element_type=jnp.float32,
        )
        return dinv * agg + dinv2 * h + b

    h1 = jnp.dot(x_ref[...], w1_ref[...],
                 preferred_element_type=jnp.float32)
    y1 = jax.nn.relu(prop(h1, b1_ref[...]))
    h2 = jnp.dot(y1, w2_ref[...],
                 preferred_element_type=jnp.float32)
    out_ref[...] = prop(h2, b2_ref[...])


def kernel(node_feature, adjacency_matrix, W1, b1, W2, b2):
    x = node_feature.astype(jnp.float32)
    if x.ndim == 3:
        x = x.reshape(-1, x.shape[-1])
    n = x.shape[0]
    out = pl.pallas_call(
        _gcn2_kernel,
        out_shape=jax.ShapeDtypeStruct((n, W2.shape[1]), jnp.float32),
    )(adjacency_matrix, x, W1, b1.reshape(1, -1), W2, b2.reshape(1, -1))
    return out.reshape(1, n, W2.shape[1])
